# baseline (speedup 1.0000x reference)
import jax
import jax.numpy as jnp
from jax.experimental import pallas as pl
from jax.experimental.pallas import tpu as pltpu

B, N, D, DO = 256, 64, 16, 16
IB = 32


def _main_body(a_ref, d4_ref, ad_ref, dd_ref, f_ref, ftf_ref, w_ref,
               b_ref, o_ref, o2_ref, g_scr, gt_scr):
    p = pl.program_id(0)

    @pl.when(p == 0)
    def _():
        w2 = w_ref[:, D:2 * D]
        for j in range(N):
            r = jnp.dot(w2, ftf_ref[j])
            gt_scr[j] = r
            g_scr[:, j, :] = r

    a = a_ref[...]
    d4 = d4_ref[...]
    o2_ref[...] = d4
    dxv = d4[:, :, 0, :]
    dyv = d4[:, :, 1, :]

    dxw = jnp.sum(a * dxv, axis=1)
    dyw = jnp.sum(a * dyv, axis=1)

    g = g_scr[...]
    w1 = w_ref[:, :D]
    w30 = w_ref[:, 2 * D:2 * D + 1]
    w31 = w_ref[:, 2 * D + 1:2 * D + 2]
    bs = (N - 1.0) * jnp.transpose(b_ref[...], (1, 0))
    for r in range(IB):
        s = jnp.sum(g * a[r][None], axis=1)
        base = (N - 1.0) * jnp.dot(w1, f_ref[r])
        g_i = gt_scr[p * IB + r]
        aii = ad_ref[r, r]
        vdx = dd_ref[r, r, 0]
        vdy = dd_ref[r, r, 1]
        dcon = w30 * dxw[r][None] + w31 * dyw[r][None]
        selfd = g_i + w30 * vdx[None] + w31 * vdy[None]
        out = base + bs + s + dcon - aii[None] * selfd
        o_ref[r] = jnp.maximum(out, 0.0)


@jax.jit
def _run(at, d4, ft, w, b2):
    out, d4c = pl.pallas_call(
        _main_body,
        grid=(N // IB,),
        in_specs=[
            pl.BlockSpec((IB, N, B), lambda p: (p, 0, 0)),
            pl.BlockSpec((IB, N, 2, B), lambda p: (p, 0, 0, 0)),
            pl.BlockSpec((IB, IB, B), lambda p: (p, p, 0)),
            pl.BlockSpec((IB, IB, 2, B), lambda p: (p, p, 0, 0)),
            pl.BlockSpec((IB, D, B), lambda p: (p, 0, 0)),
            pl.BlockSpec((N, D, B), lambda p: (0, 0, 0)),
            pl.BlockSpec((DO, 2 * D + 2), lambda p: (0, 0)),
            pl.BlockSpec((1, DO), lambda p: (0, 0)),
        ],
        scratch_shapes=[
            pltpu.VMEM((DO, N, B), jnp.float32),
            pltpu.VMEM((N, DO, B), jnp.float32),
        ],
        out_specs=[
            pl.BlockSpec((IB, DO, B), lambda p: (p, 0, 0)),
            pl.BlockSpec((IB, N, 2, B), lambda p: (p, 0, 0, 0)),
        ],
        out_shape=[
            jax.ShapeDtypeStruct((N, DO, B), jnp.float32),
            jax.ShapeDtypeStruct((N, N, 2, B), jnp.float32),
        ],
        compiler_params=pltpu.CompilerParams(
            dimension_semantics=("arbitrary",),
        ),
    )(at, d4, at, d4, ft, ft, w, b2)
    return out, d4c


def kernel(diff_vecs, agent_features, A, W, b):
    at = jnp.transpose(A, (1, 2, 0))
    d4 = jnp.transpose(diff_vecs, (1, 2, 3, 0))
    ft = jnp.transpose(agent_features, (1, 2, 0))
    b2 = b.reshape(1, DO)
    out_t, d4c = _run(at, d4, ft, W, b2)
    out = jnp.transpose(out_t, (2, 0, 1))
    diff_out = jnp.transpose(d4c, (3, 0, 1, 2))
    return (diff_out, out)


# speedup vs baseline: 1.1347x; 1.1347x over previous
import jax
import jax.numpy as jnp
from jax.experimental import pallas as pl
from jax.experimental.pallas import tpu as pltpu

B, N, D, DO = 256, 64, 16, 16
IB = 8


def _main_body(a_ref, d4_ref, ad_ref, dd_ref, f_ref, ftf_ref, w_ref,
               b_ref, o_ref, o2_ref, g_scr, gt_scr):
    p = pl.program_id(0)

    @pl.when(p == 0)
    def _():
        w2 = w_ref[:, D:2 * D]
        for j in range(N):
            r = jnp.dot(w2, ftf_ref[j])
            gt_scr[j] = r
            g_scr[:, j, :] = r

    a = a_ref[...]
    d4 = d4_ref[...]
    o2_ref[...] = d4
    dxv = d4[:, :, 0, :]
    dyv = d4[:, :, 1, :]

    dxw = jnp.sum(a * dxv, axis=1)
    dyw = jnp.sum(a * dyv, axis=1)

    g = g_scr[...]
    w1 = w_ref[:, :D]
    w30 = w_ref[:, 2 * D:2 * D + 1]
    w31 = w_ref[:, 2 * D + 1:2 * D + 2]
    bs = (N - 1.0) * jnp.transpose(b_ref[...], (1, 0))
    for r in range(IB):
        s = jnp.sum(g * a[r][None], axis=1)
        base = (N - 1.0) * jnp.dot(w1, f_ref[r])
        g_i = gt_scr[p * IB + r]
        aii = ad_ref[r, r]
        vdx = dd_ref[r, r, 0]
        vdy = dd_ref[r, r, 1]
        dcon = w30 * dxw[r][None] + w31 * dyw[r][None]
        selfd = g_i + w30 * vdx[None] + w31 * vdy[None]
        out = base + bs + s + dcon - aii[None] * selfd
        o_ref[r] = jnp.maximum(out, 0.0)


@jax.jit
def _run(at, d4, ft, w, b2):
    out, d4c = pl.pallas_call(
        _main_body,
        grid=(N // IB,),
        in_specs=[
            pl.BlockSpec((IB, N, B), lambda p: (p, 0, 0)),
            pl.BlockSpec((IB, N, 2, B), lambda p: (p, 0, 0, 0)),
            pl.BlockSpec((IB, IB, B), lambda p: (p, p, 0)),
            pl.BlockSpec((IB, IB, 2, B), lambda p: (p, p, 0, 0)),
            pl.BlockSpec((IB, D, B), lambda p: (p, 0, 0)),
            pl.BlockSpec((N, D, B), lambda p: (0, 0, 0)),
            pl.BlockSpec((DO, 2 * D + 2), lambda p: (0, 0)),
            pl.BlockSpec((1, DO), lambda p: (0, 0)),
        ],
        scratch_shapes=[
            pltpu.VMEM((DO, N, B), jnp.float32),
            pltpu.VMEM((N, DO, B), jnp.float32),
        ],
        out_specs=[
            pl.BlockSpec((IB, DO, B), lambda p: (p, 0, 0)),
            pl.BlockSpec((IB, N, 2, B), lambda p: (p, 0, 0, 0)),
        ],
        out_shape=[
            jax.ShapeDtypeStruct((N, DO, B), jnp.float32),
            jax.ShapeDtypeStruct((N, N, 2, B), jnp.float32),
        ],
        compiler_params=pltpu.CompilerParams(
            dimension_semantics=("arbitrary",),
        ),
    )(at, d4, at, d4, ft, ft, w, b2)
    return out, d4c


def kernel(diff_vecs, agent_features, A, W, b):
    at = jnp.transpose(A, (1, 2, 0))
    d4 = jnp.transpose(diff_vecs, (1, 2, 3, 0))
    ft = jnp.transpose(agent_features, (1, 2, 0))
    b2 = b.reshape(1, DO)
    out_t, d4c = _run(at, d4, ft, W, b2)
    out = jnp.transpose(out_t, (2, 0, 1))
    diff_out = jnp.transpose(d4c, (3, 0, 1, 2))
    return (diff_out, out)
